# baseline (speedup 1.0000x reference)
import functools

import jax
import jax.numpy as jnp
from jax import lax
from jax.experimental import pallas as pl
from jax.experimental.pallas import tpu as pltpu
from jax.experimental.pallas import tpu_sc as plsc

B = 4096
V = 7000
D = 100
DP = 128
M = 32


def _pad_body(w_ref, out_ref):
    out_ref[:, :D] = w_ref[...]
    out_ref[:, D:] = jnp.zeros_like(out_ref[:, D:])


@functools.lru_cache(maxsize=None)
def _make_tc_pad():
    blk = 1000
    return pl.pallas_call(
        _pad_body,
        grid=(V // blk,),
        in_specs=[pl.BlockSpec((blk, D), lambda i: (i, 0))],
        out_specs=pl.BlockSpec((blk, DP), lambda i: (i, 0)),
        out_shape=jax.ShapeDtypeStruct((V, DP), jnp.float32),
    )


@functools.lru_cache(maxsize=None)
def _make_sc_gather():
    info = plsc.get_sparse_core_info()
    nw = info.num_cores * info.num_subcores
    b_per_w = B // nw
    mesh = plsc.VectorSubcoreMesh(core_axis_name="c", subcore_axis_name="s")

    @functools.partial(
        pl.kernel,
        mesh=mesh,
        out_type=jax.ShapeDtypeStruct((B, DP), jnp.float32),
        scratch_types=[
            pltpu.VMEM((b_per_w,), jnp.int32),
            pltpu.VMEM((b_per_w, DP), jnp.float32),
            pltpu.SemaphoreType.DMA,
        ],
    )
    def gather_k(idx_hbm, table_hbm, out_hbm, idx_v, rows_v, sem):
        wid = lax.axis_index("s") * info.num_cores + lax.axis_index("c")
        base = wid * b_per_w
        pltpu.sync_copy(idx_hbm.at[pl.ds(base, b_per_w)], idx_v)
        pltpu.async_copy(table_hbm.at[idx_v], rows_v, sem).wait()
        pltpu.sync_copy(rows_v, out_hbm.at[pl.ds(base, b_per_w)])

    return gather_k


def _combine_body(rows_ref, mm_ref, out_ref):
    x = rows_ref[:, :D]
    m = jnp.max(x, axis=-1, keepdims=True)
    e = jnp.exp(x - m)
    w = e / jnp.sum(e, axis=-1, keepdims=True)
    zt = lax.dot_general(mm_ref[...], w, (((0,), (1,)), ((), ())),
                         preferred_element_type=jnp.float32)
    n = jnp.sqrt(jnp.sum(zt * zt, axis=0, keepdims=True))
    out_ref[...] = zt / jnp.maximum(n, 1e-12)


@functools.lru_cache(maxsize=None)
def _make_tc_combine():
    blk = 1024
    return pl.pallas_call(
        _combine_body,
        grid=(B // blk,),
        in_specs=[
            pl.BlockSpec((blk, DP), lambda i: (i, 0)),
            pl.BlockSpec((D, M), lambda i: (0, 0)),
        ],
        out_specs=pl.BlockSpec((M, blk), lambda i: (0, i)),
        out_shape=jax.ShapeDtypeStruct((M, B), jnp.float32),
    )


def kernel(idx, weight_embedding, main_modes):
    table = _make_tc_pad()(weight_embedding)
    rows = _make_sc_gather()(idx.astype(jnp.int32), table)
    zt = _make_tc_combine()(rows, main_modes)
    return jnp.transpose(zt)[:, None, :]


# speedup vs baseline: 1.1999x; 1.1999x over previous
import functools

import jax
import jax.numpy as jnp
from jax import lax
from jax.experimental import pallas as pl
from jax.experimental.pallas import tpu as pltpu
from jax.experimental.pallas import tpu_sc as plsc

B = 4096
V = 7000
D = 100
DP = 128
M = 32


def _pad_body(w_ref, out_ref):
    out_ref[:, :D] = w_ref[...]
    out_ref[:, D:] = jnp.zeros_like(out_ref[:, D:])


@functools.lru_cache(maxsize=None)
def _make_tc_pad():
    blk = 1000
    return pl.pallas_call(
        _pad_body,
        grid=(V // blk,),
        in_specs=[pl.BlockSpec((blk, D), lambda i: (i, 0))],
        out_specs=pl.BlockSpec((blk, DP), lambda i: (i, 0)),
        out_shape=jax.ShapeDtypeStruct((V, DP), jnp.float32),
    )


@functools.lru_cache(maxsize=None)
def _make_sc_gather():
    info = plsc.get_sparse_core_info()
    nw = info.num_cores * info.num_subcores
    b_per_w = B // nw
    mesh = plsc.VectorSubcoreMesh(core_axis_name="c", subcore_axis_name="s")

    @functools.partial(
        pl.kernel,
        mesh=mesh,
        out_type=jax.ShapeDtypeStruct((B, DP), jnp.float32),
        scratch_types=[
            pltpu.VMEM((b_per_w,), jnp.int32),
            pltpu.VMEM((b_per_w, DP), jnp.float32),
            pltpu.SemaphoreType.DMA,
        ],
    )
    def gather_k(idx_hbm, table_hbm, out_hbm, idx_v, rows_v, sem):
        wid = lax.axis_index("s") * info.num_cores + lax.axis_index("c")
        base = wid * b_per_w
        pltpu.sync_copy(idx_hbm.at[pl.ds(base, b_per_w)], idx_v)
        pltpu.async_copy(table_hbm.at[idx_v], rows_v, sem).wait()
        pltpu.sync_copy(rows_v, out_hbm.at[pl.ds(base, b_per_w)])

    return gather_k


def _combine_body(rows_ref, mm_ref, out_ref):
    x = rows_ref[:, :D]
    m = jnp.max(x, axis=-1, keepdims=True)
    e = jnp.exp(x - m)
    w = e / jnp.sum(e, axis=-1, keepdims=True)
    zt = lax.dot_general(mm_ref[...], w, (((0,), (1,)), ((), ())),
                         preferred_element_type=jnp.float32)
    n = jnp.sqrt(jnp.sum(zt * zt, axis=0, keepdims=True))
    out_ref[...] = zt / jnp.maximum(n, 1e-12)


@functools.lru_cache(maxsize=None)
def _make_tc_combine():
    blk = 1024
    return pl.pallas_call(
        _combine_body,
        grid=(B // blk,),
        in_specs=[
            pl.BlockSpec((blk, DP), lambda i: (i, 0)),
            pl.BlockSpec((D, M), lambda i: (0, 0)),
        ],
        out_specs=pl.BlockSpec((M, blk), lambda i: (0, i)),
        out_shape=jax.ShapeDtypeStruct((M, B), jnp.float32),
    )


def kernel(idx, weight_embedding, main_modes):
    table = jnp.pad(weight_embedding, ((0, 0), (0, DP - D)))
    rows = _make_sc_gather()(idx.astype(jnp.int32), table)
    zt = _make_tc_combine()(rows, main_modes)
    return jnp.transpose(zt)[:, None, :]
